# Initial kernel scaffold; baseline (speedup 1.0000x reference)
import jax
import jax.numpy as jnp
from jax.experimental import pallas as pl


def kernel(win_seq, win_pcc, global_adjs, patch_adjs, embeddings, fgc_w, fgc_b, w_t, w_c, ln_gamma, ln_beta, router_w, router_b, expert_w, l1_w, l1_b, l2_w, l2_b, l3_w, l3_b, l4_w, l4_b):
    raise NotImplementedError("write your pallas kernel here")


# speedup vs baseline: 5.4625x; 5.4625x over previous
import jax
import jax.numpy as jnp
import numpy as np
from jax.experimental import pallas as pl

B, NW, R, WS, E, LAYERS, NE, TOPK = 64, 17, 116, 10, 16, 5, 4, 2
D = NW * WS
N = R * WS
F = N // 2 + 1
FP = 640
NWP = 20
COLS = NWP * 32
HCOLS = NWP * 16
LAMBD = 0.01

_INTERPRET = False


def _dot_hi(a, b):
    return jax.lax.dot_general(a, b, (((a.ndim - 1,), (0,)), ((), ())),
                               precision=jax.lax.Precision.HIGHEST,
                               preferred_element_type=jnp.float32)


def _dot_bf(a, b):
    return jax.lax.dot_general(a, b, (((a.ndim - 1,), (0,)), ((), ())),
                               precision=jax.lax.Precision.DEFAULT,
                               preferred_element_type=jnp.float32)


def _np_dft_bases():
    n = np.arange(N)[:, None].astype(np.float64)
    f = np.arange(FP)[None, :].astype(np.float64)
    ang = 2.0 * np.pi * n * f / N
    scale = 1.0 / np.sqrt(N)
    valid = (f < F).astype(np.float64)
    c = np.cos(ang) * scale * valid
    s = -np.sin(ang) * scale * valid
    w = (np.where((f == 0) | (f == F - 1), 1.0, 2.0) * valid)
    cit = np.cos(ang) * scale * w
    sit = -np.sin(ang) * scale * w
    return (np.asarray(c, np.float32), np.asarray(s, np.float32),
            np.asarray(cit, np.float32), np.asarray(sit, np.float32))

_C_FWD, _S_FWD, _CIT, _SIT = _np_dft_bases()


def _np_sel():
    sela = np.zeros((NW, COLS), np.float32)
    selb = np.zeros((NW, COLS), np.float32)
    for w in range(NW):
        for j in range(E):
            sela[w, w * 32 + j] = 1.0
            selb[w, w * 32 + 16 + j] = 1.0
    return sela, selb

_SELA, _SELB = _np_sel()


def _p1_body(x_ref, c_ref, s_ref, fre_ref, fim_ref):
    x = x_ref[...]
    fre_ref[...] = _dot_hi(x, c_ref[...])
    fim_ref[...] = _dot_hi(x, s_ref[...])


def _p1(xw):
    tm = 136
    grid = (NW * B) // tm
    return pl.pallas_call(
        _p1_body,
        grid=(grid,),
        in_specs=[pl.BlockSpec((tm, N), lambda i: (i, 0)),
                  pl.BlockSpec((N, FP), lambda i: (0, 0)),
                  pl.BlockSpec((N, FP), lambda i: (0, 0))],
        out_specs=[pl.BlockSpec((tm, FP), lambda i: (i, 0)),
                   pl.BlockSpec((tm, FP), lambda i: (i, 0))],
        out_shape=[jax.ShapeDtypeStruct((NW * B, FP), jnp.float32),
                   jax.ShapeDtypeStruct((NW * B, FP), jnp.float32)],
        interpret=_INTERPRET,
    )(xw, _C_FWD, _S_FWD)


def _p2_body(fre_ref, fim_ref, sela_ref, selb_ref, embrow_ref, wbig_ref,
             bbig_ref, wlre_ref, wlim_ref, blre_ref, blim_ref,
             are_ref, aim_ref):
    sel = _dot_hi(fre_ref[...], sela_ref[...]) + _dot_hi(fim_ref[...],
                                                         selb_ref[...])
    a = sel * embrow_ref[...]
    for l in range(LAYERS - 1):
        parts = [_dot_bf(a[:, 128 * g:128 * (g + 1)], wbig_ref[l, g])
                 for g in range(5)]
        a = jax.nn.relu(jnp.concatenate(parts, axis=1) + bbig_ref[l:l + 1, :]
                        - LAMBD)
    re_parts = [_dot_bf(a[:, 128 * g:128 * (g + 1)], wlre_ref[g])
                for g in range(5)]
    im_parts = [_dot_bf(a[:, 128 * g:128 * (g + 1)], wlim_ref[g])
                for g in range(5)]
    are_ref[...] = jax.nn.relu(jnp.concatenate(re_parts, axis=1)
                               + blre_ref[...] - LAMBD)
    aim_ref[...] = jax.nn.relu(jnp.concatenate(im_parts, axis=1)
                               + blim_ref[...] - LAMBD)


def _p2(fre_t, fim_t, embrow, wbig, bbig, wlre, wlim, blre, blim):
    m = B * FP
    tm = 2048
    grid = m // tm
    return pl.pallas_call(
        _p2_body,
        grid=(grid,),
        in_specs=[pl.BlockSpec((tm, NW), lambda i: (i, 0)),
                  pl.BlockSpec((tm, NW), lambda i: (i, 0)),
                  pl.BlockSpec((NW, COLS), lambda i: (0, 0)),
                  pl.BlockSpec((NW, COLS), lambda i: (0, 0)),
                  pl.BlockSpec((1, COLS), lambda i: (0, 0)),
                  pl.BlockSpec((LAYERS - 1, 5, 128, 128),
                               lambda i: (0, 0, 0, 0)),
                  pl.BlockSpec((LAYERS - 1, COLS), lambda i: (0, 0)),
                  pl.BlockSpec((5, 128, 64), lambda i: (0, 0, 0)),
                  pl.BlockSpec((5, 128, 64), lambda i: (0, 0, 0)),
                  pl.BlockSpec((1, HCOLS), lambda i: (0, 0)),
                  pl.BlockSpec((1, HCOLS), lambda i: (0, 0))],
        out_specs=[pl.BlockSpec((tm, HCOLS), lambda i: (i, 0)),
                   pl.BlockSpec((tm, HCOLS), lambda i: (i, 0))],
        out_shape=[jax.ShapeDtypeStruct((m, HCOLS), jnp.float32),
                   jax.ShapeDtypeStruct((m, HCOLS), jnp.float32)],
        interpret=_INTERPRET,
    )(fre_t, fim_t, _SELA, _SELB, embrow, wbig, bbig, wlre, wlim, blre, blim)


def _p3_body(are_ref, aim_ref, cit_ref, sit_ref, q_ref, tok_ref):
    y = _dot_hi(cit_ref[...], are_ref[0]) + _dot_hi(sit_ref[...], aim_ref[0])
    tok_ref[0] = _dot_bf(y, q_ref[...])


def _p3(are3, aim3, q):
    return pl.pallas_call(
        _p3_body,
        grid=(B,),
        in_specs=[pl.BlockSpec((1, FP, HCOLS), lambda b: (b, 0, 0)),
                  pl.BlockSpec((1, FP, HCOLS), lambda b: (b, 0, 0)),
                  pl.BlockSpec((N, FP), lambda b: (0, 0)),
                  pl.BlockSpec((N, FP), lambda b: (0, 0)),
                  pl.BlockSpec((HCOLS, 32), lambda b: (0, 0))],
        out_specs=pl.BlockSpec((1, N, 32), lambda b: (b, 0, 0)),
        out_shape=jax.ShapeDtypeStruct((B, N, 32), jnp.float32),
        interpret=_INTERPRET,
    )(are3, aim3, _CIT, _SIT, q)


def _p4_body(tok_ref, adj_ref, v_ref, lng_ref, lnb_ref, rw_ref, rb_ref,
             wall_ref, out_ref):
    tokb = tok_ref[0] * v_ref[...]
    mu = jnp.mean(tokb, axis=-1, keepdims=True)
    var = jnp.mean((tokb - mu) ** 2, axis=-1, keepdims=True)
    tn = (tokb - mu) / jnp.sqrt(var + 1e-5) * lng_ref[...] + lnb_ref[...]
    lg = _dot_bf(tn, rw_ref[...]) + rb_ref[...]
    cols = [lg[:, e:e + 1] for e in range(NE)]
    m = jnp.maximum(jnp.maximum(cols[0], cols[1]),
                    jnp.maximum(cols[2], cols[3]))
    ge = []
    for e in range(NE):
        rank = jnp.zeros_like(cols[e])
        for j in range(NE):
            if j == e:
                continue
            gt = (cols[j] > cols[e]) | ((cols[j] == cols[e]) & (j < e))
            rank = rank + gt.astype(jnp.float32)
        sel = (rank < TOPK).astype(jnp.float32)
        ge.append(sel * jnp.exp(cols[e] - m))
    den = ge[0] + ge[1] + ge[2] + ge[3]
    s_all = _dot_bf(tokb, wall_ref[...])
    z = jax.nn.relu(_dot_bf(adj_ref[0], s_all))
    moe = jnp.zeros_like(tokb)
    for e in range(NE):
        moe = moe + (ge[e] / den) * z[:, e * D:(e + 1) * D]
    out_ref[0] = tokb + moe


def _p4(tok_u, adjs, v170, lng, lnb, rw, rb, wall):
    return pl.pallas_call(
        _p4_body,
        grid=(B,),
        in_specs=[pl.BlockSpec((1, R, D), lambda b: (b, 0, 0)),
                  pl.BlockSpec((1, R, R), lambda b: (b, 0, 0)),
                  pl.BlockSpec((1, D), lambda b: (0, 0)),
                  pl.BlockSpec((1, D), lambda b: (0, 0)),
                  pl.BlockSpec((1, D), lambda b: (0, 0)),
                  pl.BlockSpec((D, NE), lambda b: (0, 0)),
                  pl.BlockSpec((1, NE), lambda b: (0, 0)),
                  pl.BlockSpec((D, NE * D), lambda b: (0, 0))],
        out_specs=pl.BlockSpec((1, R, D), lambda b: (b, 0, 0)),
        out_shape=jax.ShapeDtypeStruct((B, R, D), jnp.float32),
        interpret=_INTERPRET,
    )(tok_u, adjs, v170, lng, lnb, rw, rb, wall)


_INV_BN = np.float32(1.0) / np.sqrt(np.float32(1.0 + 1e-5))


def _p5_body(h_ref, w_ref, b_ref, o_ref):
    o_ref[...] = jax.nn.relu(_dot_bf(h_ref[...], w_ref[...])
                             + b_ref[...]) * _INV_BN


def _p5(h, l1_w, l1_b):
    tn = 128
    return pl.pallas_call(
        _p5_body,
        grid=(512 // tn,),
        in_specs=[pl.BlockSpec((B, R * D), lambda n: (0, 0)),
                  pl.BlockSpec((R * D, tn), lambda n: (0, n)),
                  pl.BlockSpec((1, tn), lambda n: (0, n))],
        out_specs=pl.BlockSpec((B, tn), lambda n: (0, n)),
        out_shape=jax.ShapeDtypeStruct((B, 512), jnp.float32),
        interpret=_INTERPRET,
    )(h, l1_w, l1_b)


def _p6_body(h_ref, w2_ref, b2_ref, w3_ref, b3_ref, w4_ref, b4_ref, o_ref):
    h = h_ref[...]
    h = jax.nn.relu(_dot_bf(h, w2_ref[...]) + b2_ref[...]) * _INV_BN
    h = jax.nn.relu(_dot_bf(h, w3_ref[...]) + b3_ref[...]) * _INV_BN
    o_ref[...] = _dot_bf(h, w4_ref[...]) + b4_ref[...]


def _p6(h1, l2_w, l2_b, l3_w, l3_b, l4_w, l4_b):
    return pl.pallas_call(
        _p6_body,
        grid=(1,),
        in_specs=[pl.BlockSpec((B, 512), lambda i: (0, 0)),
                  pl.BlockSpec((512, 256), lambda i: (0, 0)),
                  pl.BlockSpec((1, 256), lambda i: (0, 0)),
                  pl.BlockSpec((256, 128), lambda i: (0, 0)),
                  pl.BlockSpec((1, 128), lambda i: (0, 0)),
                  pl.BlockSpec((128, 2), lambda i: (0, 0)),
                  pl.BlockSpec((1, 2), lambda i: (0, 0))],
        out_specs=pl.BlockSpec((B, 2), lambda i: (0, 0)),
        out_shape=jax.ShapeDtypeStruct((B, 2), jnp.float32),
        interpret=_INTERPRET,
    )(h1, l2_w, l2_b, l3_w, l3_b, l4_w, l4_b)


def kernel(win_seq, win_pcc, global_adjs, patch_adjs, embeddings, fgc_w, fgc_b,
           w_t, w_c, ln_gamma, ln_beta, router_w, router_b, expert_w,
           l1_w, l1_b, l2_w, l2_b, l3_w, l3_b, l4_w, l4_b):
    emb = embeddings[0]

    embrow = jnp.tile(jnp.concatenate([emb, emb]), NWP).reshape(1, COLS)

    wr = fgc_w[:, :LAYERS - 1, 0]
    wi = fgc_w[:, :LAYERS - 1, 1]
    blk = jnp.concatenate(
        [jnp.concatenate([wr, wi], axis=3),
         jnp.concatenate([-wi, wr], axis=3)], axis=2)
    blk = jnp.pad(blk, ((0, NWP - NW), (0, 0), (0, 0), (0, 0)))
    blk = blk.transpose(1, 0, 2, 3).reshape(LAYERS - 1, 5, 4, 32, 32)
    eye4 = jnp.eye(4, dtype=jnp.float32)
    wbig = (eye4[None, None, :, None, :, None] *
            blk[:, :, :, :, None, :]).reshape(LAYERS - 1, 5, 128, 128)
    bbig = jnp.pad(
        fgc_b[:, :LAYERS - 1].transpose(1, 0, 2, 3).reshape(LAYERS - 1,
                                                            NW * 32),
        ((0, 0), (0, COLS - NW * 32)))

    wr5 = fgc_w[:, LAYERS - 1, 0]
    wi5 = fgc_w[:, LAYERS - 1, 1]
    blk_re = jnp.pad(jnp.concatenate([wr5, -wi5], axis=1),
                     ((0, NWP - NW), (0, 0), (0, 0)))
    blk_im = jnp.pad(jnp.concatenate([wi5, wr5], axis=1),
                     ((0, NWP - NW), (0, 0), (0, 0)))
    wlre = (eye4[None, :, None, :, None] *
            blk_re.reshape(5, 4, 32, 16)[:, :, :, None, :]).reshape(5, 128, 64)
    wlim = (eye4[None, :, None, :, None] *
            blk_im.reshape(5, 4, 32, 16)[:, :, :, None, :]).reshape(5, 128, 64)
    blre = jnp.pad(fgc_b[:, LAYERS - 1, 0].reshape(1, NW * E),
                   ((0, 0), (0, HCOLS - NW * E)))
    blim = jnp.pad(fgc_b[:, LAYERS - 1, 1].reshape(1, NW * E),
                   ((0, 0), (0, HCOLS - NW * E)))

    wcs = jax.nn.softmax(w_c, axis=-1)
    eye_w = jnp.eye(32, dtype=jnp.float32)[:NW]
    q = jnp.pad((wcs[:, :, None] * eye_w[:, None, :]).reshape(NW * E, 32),
                ((0, HCOLS - NW * E), (0, 0)))

    wts = jax.nn.softmax(w_t, axis=-1)
    v170 = wts.reshape(1, D)
    wall = expert_w.transpose(1, 0, 2).reshape(D, NE * D)

    xw = win_seq.transpose(1, 0, 2, 3).reshape(NW * B, N)
    fre, fim = _p1(xw)
    fre_t = fre.reshape(NW, B, FP).transpose(1, 2, 0).reshape(B * FP, NW)
    fim_t = fim.reshape(NW, B, FP).transpose(1, 2, 0).reshape(B * FP, NW)

    a5re, a5im = _p2(fre_t, fim_t, embrow, wbig, bbig, wlre, wlim, blre, blim)
    are3 = a5re.reshape(B, FP, HCOLS)
    aim3 = a5im.reshape(B, FP, HCOLS)

    tokpre = _p3(are3, aim3, q)
    tok_u = tokpre[:, :, :NW].reshape(B, R, WS, NW).transpose(0, 1, 3, 2)
    tok_u = tok_u.reshape(B, R, D)

    h3 = _p4(tok_u, global_adjs, v170, ln_gamma.reshape(1, D),
             ln_beta.reshape(1, D), router_w, router_b.reshape(1, NE), wall)
    h = h3.reshape(B, R * D)

    h1 = _p5(h, l1_w, l1_b.reshape(1, 512))
    return _p6(h1, l2_w, l2_b.reshape(1, 256), l3_w, l3_b.reshape(1, 128),
               l4_w, l4_b.reshape(1, 2))


# speedup vs baseline: 5.4647x; 1.0004x over previous
import jax
import jax.numpy as jnp
import numpy as np
from jax.experimental import pallas as pl

B, NW, R, WS, E, LAYERS, NE, TOPK = 64, 17, 116, 10, 16, 5, 4, 2
D = NW * WS
N = R * WS
F = N // 2 + 1
FP = 640
NWP = 20
COLS = NWP * 32
HCOLS = NWP * 16
LAMBD = 0.01

_INTERPRET = False


def _dot_hi(a, b):
    return jax.lax.dot_general(a, b, (((a.ndim - 1,), (0,)), ((), ())),
                               precision=jax.lax.Precision.HIGHEST,
                               preferred_element_type=jnp.float32)


def _dot_bf(a, b):
    return jax.lax.dot_general(a, b, (((a.ndim - 1,), (0,)), ((), ())),
                               precision=jax.lax.Precision.DEFAULT,
                               preferred_element_type=jnp.float32)


def _np_dft_bases():
    n = np.arange(N)[:, None].astype(np.float64)
    f = np.arange(FP)[None, :].astype(np.float64)
    ang = 2.0 * np.pi * n * f / N
    scale = 1.0 / np.sqrt(N)
    valid = (f < F).astype(np.float64)
    c = np.cos(ang) * scale * valid
    s = -np.sin(ang) * scale * valid
    w = (np.where((f == 0) | (f == F - 1), 1.0, 2.0) * valid)
    cit = np.cos(ang) * scale * w
    sit = -np.sin(ang) * scale * w
    return (np.asarray(c, np.float32), np.asarray(s, np.float32),
            np.asarray(cit, np.float32), np.asarray(sit, np.float32))

_C_FWD, _S_FWD, _CIT, _SIT = _np_dft_bases()


def _np_sel():
    sela = np.zeros((NW, COLS), np.float32)
    selb = np.zeros((NW, COLS), np.float32)
    for w in range(NW):
        for j in range(E):
            sela[w, w * 32 + j] = 1.0
            selb[w, w * 32 + 16 + j] = 1.0
    return sela, selb

_SELA, _SELB = _np_sel()


def _p1_body(x_ref, c_ref, s_ref, fre_ref, fim_ref):
    x = x_ref[...]
    fre_ref[...] = _dot_hi(x, c_ref[...])
    fim_ref[...] = _dot_hi(x, s_ref[...])


def _p1(xw):
    tm = 136
    grid = (NW * B) // tm
    return pl.pallas_call(
        _p1_body,
        grid=(grid,),
        in_specs=[pl.BlockSpec((tm, N), lambda i: (i, 0)),
                  pl.BlockSpec((N, FP), lambda i: (0, 0)),
                  pl.BlockSpec((N, FP), lambda i: (0, 0))],
        out_specs=[pl.BlockSpec((tm, FP), lambda i: (i, 0)),
                   pl.BlockSpec((tm, FP), lambda i: (i, 0))],
        out_shape=[jax.ShapeDtypeStruct((NW * B, FP), jnp.float32),
                   jax.ShapeDtypeStruct((NW * B, FP), jnp.float32)],
        interpret=_INTERPRET,
    )(xw, _C_FWD, _S_FWD)


def _p2_body(fre_ref, fim_ref, sela_ref, selb_ref, embrow_ref, wbig_ref,
             bbig_ref, wlre_ref, wlim_ref, blre_ref, blim_ref,
             are_ref, aim_ref):
    fre = fre_ref[...]
    fim = fim_ref[...]
    sela = sela_ref[...]
    selb = selb_ref[...]
    embrow = embrow_ref[...]
    bbig = bbig_ref[...]
    for g in range(5):
        c0, c1 = 128 * g, 128 * (g + 1)
        a = (_dot_hi(fre, sela[:, c0:c1]) + _dot_hi(fim, selb[:, c0:c1]))
        a = a * embrow[:, c0:c1]
        for l in range(LAYERS - 1):
            a = jax.nn.relu(_dot_bf(a, wbig_ref[l, g])
                            + bbig[l:l + 1, c0:c1] - LAMBD)
        h0, h1 = 64 * g, 64 * (g + 1)
        are_ref[:, h0:h1] = jax.nn.relu(_dot_bf(a, wlre_ref[g])
                                        + blre_ref[:, h0:h1] - LAMBD)
        aim_ref[:, h0:h1] = jax.nn.relu(_dot_bf(a, wlim_ref[g])
                                        + blim_ref[:, h0:h1] - LAMBD)


def _p2(fre_t, fim_t, embrow, wbig, bbig, wlre, wlim, blre, blim):
    m = B * FP
    tm = 2048
    grid = m // tm
    return pl.pallas_call(
        _p2_body,
        grid=(grid,),
        in_specs=[pl.BlockSpec((tm, NW), lambda i: (i, 0)),
                  pl.BlockSpec((tm, NW), lambda i: (i, 0)),
                  pl.BlockSpec((NW, COLS), lambda i: (0, 0)),
                  pl.BlockSpec((NW, COLS), lambda i: (0, 0)),
                  pl.BlockSpec((1, COLS), lambda i: (0, 0)),
                  pl.BlockSpec((LAYERS - 1, 5, 128, 128),
                               lambda i: (0, 0, 0, 0)),
                  pl.BlockSpec((LAYERS - 1, COLS), lambda i: (0, 0)),
                  pl.BlockSpec((5, 128, 64), lambda i: (0, 0, 0)),
                  pl.BlockSpec((5, 128, 64), lambda i: (0, 0, 0)),
                  pl.BlockSpec((1, HCOLS), lambda i: (0, 0)),
                  pl.BlockSpec((1, HCOLS), lambda i: (0, 0))],
        out_specs=[pl.BlockSpec((tm, HCOLS), lambda i: (i, 0)),
                   pl.BlockSpec((tm, HCOLS), lambda i: (i, 0))],
        out_shape=[jax.ShapeDtypeStruct((m, HCOLS), jnp.float32),
                   jax.ShapeDtypeStruct((m, HCOLS), jnp.float32)],
        interpret=_INTERPRET,
    )(fre_t, fim_t, _SELA, _SELB, embrow, wbig, bbig, wlre, wlim, blre, blim)


NH = F
KP = 584


def _p3_body(are_ref, aim_ref, cut_ref, svt_ref, q_ref, top_ref, bot_ref):
    u = _dot_hi(cut_ref[...], are_ref[0][:KP, :])
    v = _dot_hi(svt_ref[...], aim_ref[0][:KP, :])
    q = q_ref[...]
    top_ref[0] = _dot_bf(u + v, q)
    bot_ref[0] = _dot_bf(u - v, q)


def _p3(are3, aim3, q):
    return pl.pallas_call(
        _p3_body,
        grid=(B,),
        in_specs=[pl.BlockSpec((1, FP, HCOLS), lambda b: (b, 0, 0)),
                  pl.BlockSpec((1, FP, HCOLS), lambda b: (b, 0, 0)),
                  pl.BlockSpec((NH, KP), lambda b: (0, 0)),
                  pl.BlockSpec((NH, KP), lambda b: (0, 0)),
                  pl.BlockSpec((HCOLS, 32), lambda b: (0, 0))],
        out_specs=[pl.BlockSpec((1, NH, 32), lambda b: (b, 0, 0)),
                   pl.BlockSpec((1, NH, 32), lambda b: (b, 0, 0))],
        out_shape=[jax.ShapeDtypeStruct((B, NH, 32), jnp.float32),
                   jax.ShapeDtypeStruct((B, NH, 32), jnp.float32)],
        interpret=_INTERPRET,
    )(are3, aim3, _CIT[:NH, :KP], _SIT[:NH, :KP], q)


def _p4_body(tok_ref, adj_ref, v_ref, lng_ref, lnb_ref, rw_ref, rb_ref,
             wall_ref, out_ref):
    tokb = tok_ref[0] * v_ref[...]
    mu = jnp.mean(tokb, axis=-1, keepdims=True)
    var = jnp.mean((tokb - mu) ** 2, axis=-1, keepdims=True)
    tn = (tokb - mu) / jnp.sqrt(var + 1e-5) * lng_ref[...] + lnb_ref[...]
    lg = _dot_bf(tn, rw_ref[...]) + rb_ref[...]
    cols = [lg[:, e:e + 1] for e in range(NE)]
    m = jnp.maximum(jnp.maximum(cols[0], cols[1]),
                    jnp.maximum(cols[2], cols[3]))
    ge = []
    for e in range(NE):
        rank = jnp.zeros_like(cols[e])
        for j in range(NE):
            if j == e:
                continue
            gt = (cols[j] > cols[e]) | ((cols[j] == cols[e]) & (j < e))
            rank = rank + gt.astype(jnp.float32)
        sel = (rank < TOPK).astype(jnp.float32)
        ge.append(sel * jnp.exp(cols[e] - m))
    den = ge[0] + ge[1] + ge[2] + ge[3]
    s_all = _dot_bf(tokb, wall_ref[...])
    z = jax.nn.relu(_dot_bf(adj_ref[0], s_all))
    moe = jnp.zeros_like(tokb)
    for e in range(NE):
        moe = moe + (ge[e] / den) * z[:, e * D:(e + 1) * D]
    out_ref[0] = tokb + moe


def _p4(tok_u, adjs, v170, lng, lnb, rw, rb, wall):
    return pl.pallas_call(
        _p4_body,
        grid=(B,),
        in_specs=[pl.BlockSpec((1, R, D), lambda b: (b, 0, 0)),
                  pl.BlockSpec((1, R, R), lambda b: (b, 0, 0)),
                  pl.BlockSpec((1, D), lambda b: (0, 0)),
                  pl.BlockSpec((1, D), lambda b: (0, 0)),
                  pl.BlockSpec((1, D), lambda b: (0, 0)),
                  pl.BlockSpec((D, NE), lambda b: (0, 0)),
                  pl.BlockSpec((1, NE), lambda b: (0, 0)),
                  pl.BlockSpec((D, NE * D), lambda b: (0, 0))],
        out_specs=pl.BlockSpec((1, R, D), lambda b: (b, 0, 0)),
        out_shape=jax.ShapeDtypeStruct((B, R, D), jnp.float32),
        interpret=_INTERPRET,
    )(tok_u, adjs, v170, lng, lnb, rw, rb, wall)


_INV_BN = np.float32(1.0) / np.sqrt(np.float32(1.0 + 1e-5))


def _p5_body(h_ref, w_ref, b_ref, o_ref):
    o_ref[...] = jax.nn.relu(_dot_bf(h_ref[...], w_ref[...])
                             + b_ref[...]) * _INV_BN


def _p5(h, l1_w, l1_b):
    tn = 128
    return pl.pallas_call(
        _p5_body,
        grid=(512 // tn,),
        in_specs=[pl.BlockSpec((B, R * D), lambda n: (0, 0)),
                  pl.BlockSpec((R * D, tn), lambda n: (0, n)),
                  pl.BlockSpec((1, tn), lambda n: (0, n))],
        out_specs=pl.BlockSpec((B, tn), lambda n: (0, n)),
        out_shape=jax.ShapeDtypeStruct((B, 512), jnp.float32),
        interpret=_INTERPRET,
    )(h, l1_w, l1_b)


def _p6_body(h_ref, w2_ref, b2_ref, w3_ref, b3_ref, w4_ref, b4_ref, o_ref):
    h = h_ref[...]
    h = jax.nn.relu(_dot_bf(h, w2_ref[...]) + b2_ref[...]) * _INV_BN
    h = jax.nn.relu(_dot_bf(h, w3_ref[...]) + b3_ref[...]) * _INV_BN
    o_ref[...] = _dot_bf(h, w4_ref[...]) + b4_ref[...]


def _p6(h1, l2_w, l2_b, l3_w, l3_b, l4_w, l4_b):
    return pl.pallas_call(
        _p6_body,
        grid=(1,),
        in_specs=[pl.BlockSpec((B, 512), lambda i: (0, 0)),
                  pl.BlockSpec((512, 256), lambda i: (0, 0)),
                  pl.BlockSpec((1, 256), lambda i: (0, 0)),
                  pl.BlockSpec((256, 128), lambda i: (0, 0)),
                  pl.BlockSpec((1, 128), lambda i: (0, 0)),
                  pl.BlockSpec((128, 2), lambda i: (0, 0)),
                  pl.BlockSpec((1, 2), lambda i: (0, 0))],
        out_specs=pl.BlockSpec((B, 2), lambda i: (0, 0)),
        out_shape=jax.ShapeDtypeStruct((B, 2), jnp.float32),
        interpret=_INTERPRET,
    )(h1, l2_w, l2_b, l3_w, l3_b, l4_w, l4_b)


def kernel(win_seq, win_pcc, global_adjs, patch_adjs, embeddings, fgc_w, fgc_b,
           w_t, w_c, ln_gamma, ln_beta, router_w, router_b, expert_w,
           l1_w, l1_b, l2_w, l2_b, l3_w, l3_b, l4_w, l4_b):
    emb = embeddings[0]

    embrow = jnp.tile(jnp.concatenate([emb, emb]), NWP).reshape(1, COLS)

    wr = fgc_w[:, :LAYERS - 1, 0]
    wi = fgc_w[:, :LAYERS - 1, 1]
    blk = jnp.concatenate(
        [jnp.concatenate([wr, wi], axis=3),
         jnp.concatenate([-wi, wr], axis=3)], axis=2)
    blk = jnp.pad(blk, ((0, NWP - NW), (0, 0), (0, 0), (0, 0)))
    blk = blk.transpose(1, 0, 2, 3).reshape(LAYERS - 1, 5, 4, 32, 32)
    eye4 = jnp.eye(4, dtype=jnp.float32)
    wbig = (eye4[None, None, :, None, :, None] *
            blk[:, :, :, :, None, :]).reshape(LAYERS - 1, 5, 128, 128)
    bbig = jnp.pad(
        fgc_b[:, :LAYERS - 1].transpose(1, 0, 2, 3).reshape(LAYERS - 1,
                                                            NW * 32),
        ((0, 0), (0, COLS - NW * 32)))

    wr5 = fgc_w[:, LAYERS - 1, 0]
    wi5 = fgc_w[:, LAYERS - 1, 1]
    blk_re = jnp.pad(jnp.concatenate([wr5, -wi5], axis=1),
                     ((0, NWP - NW), (0, 0), (0, 0)))
    blk_im = jnp.pad(jnp.concatenate([wi5, wr5], axis=1),
                     ((0, NWP - NW), (0, 0), (0, 0)))
    wlre = (eye4[None, :, None, :, None] *
            blk_re.reshape(5, 4, 32, 16)[:, :, :, None, :]).reshape(5, 128, 64)
    wlim = (eye4[None, :, None, :, None] *
            blk_im.reshape(5, 4, 32, 16)[:, :, :, None, :]).reshape(5, 128, 64)
    blre = jnp.pad(fgc_b[:, LAYERS - 1, 0].reshape(1, NW * E),
                   ((0, 0), (0, HCOLS - NW * E)))
    blim = jnp.pad(fgc_b[:, LAYERS - 1, 1].reshape(1, NW * E),
                   ((0, 0), (0, HCOLS - NW * E)))

    wcs = jax.nn.softmax(w_c, axis=-1)
    eye_w = jnp.eye(32, dtype=jnp.float32)[:NW]
    q = jnp.pad((wcs[:, :, None] * eye_w[:, None, :]).reshape(NW * E, 32),
                ((0, HCOLS - NW * E), (0, 0)))

    wts = jax.nn.softmax(w_t, axis=-1)
    v170 = wts.reshape(1, D)
    wall = expert_w.transpose(1, 0, 2).reshape(D, NE * D)

    xw = win_seq.transpose(1, 0, 2, 3).reshape(NW * B, N)
    fre, fim = _p1(xw)
    fre_t = fre.reshape(NW, B, FP).transpose(1, 2, 0).reshape(B * FP, NW)
    fim_t = fim.reshape(NW, B, FP).transpose(1, 2, 0).reshape(B * FP, NW)

    a5re, a5im = _p2(fre_t, fim_t, embrow, wbig, bbig, wlre, wlim, blre, blim)
    are3 = a5re.reshape(B, FP, HCOLS)
    aim3 = a5im.reshape(B, FP, HCOLS)

    top, bot = _p3(are3, aim3, q)
    tokpre = jnp.concatenate([top, bot[:, 579:0:-1, :]], axis=1)
    tok_u = tokpre[:, :, :NW].reshape(B, R, WS, NW).transpose(0, 1, 3, 2)
    tok_u = tok_u.reshape(B, R, D)

    h3 = _p4(tok_u, global_adjs, v170, ln_gamma.reshape(1, D),
             ln_beta.reshape(1, D), router_w, router_b.reshape(1, NE), wall)
    h = h3.reshape(B, R * D)

    h1 = _p5(h, l1_w, l1_b.reshape(1, 512))
    return _p6(h1, l2_w, l2_b.reshape(1, 256), l3_w, l3_b.reshape(1, 128),
               l4_w, l4_b.reshape(1, 2))
